# baseline (device time: 113582 ns/iter reference)
import jax
import jax.numpy as jnp
from jax import lax
from jax.experimental import pallas as pl
from jax.experimental.pallas import tpu as pltpu

N_DEV = 8
B, SQ, SKV = 2, 512, 512
HQ_LOC, DH = 8, 64
HD = HQ_LOC * DH
EMB = 768
ROWS = B * SQ
R = ROWS // N_DEV
NG = 3
COLS = EMB // NG
QT = 128

RS_SCHED = (
    ((0, 4, ((0, 4),)), (4, 2, ((0, 2),)), (6, 1, ((0, 1),))),
    ((0, 2, ((0, 2), (4, 2))), (2, 1, ((0, 1), (4, 1))), (3, 4, ((0, 1),))),
    ((0, 1, ((0, 1), (2, 1), (4, 1), (6, 1))), (1, 4, ((0, 1), (2, 1))),
     (5, 2, ((0, 1),))),
)
AG_SCHED = (
    ((7, 1, ((0, 1),)), (6, 2, ((0, 2),)), (4, 4, ((0, 4),))),
    ((7, 4, ((0, 1),)), (3, 1, ((0, 1), (4, 1))), (2, 2, ((0, 2), (4, 2)))),
    ((7, 2, ((0, 1),)), (5, 4, ((0, 1), (2, 1))),
     (1, 1, ((0, 1), (2, 1), (4, 1), (6, 1)))),
)
RBUF_BASE = (0, 4, 6)
N_RDMA = 15


def _body(x_ref, wq_ref, k_ref, v_ref, wo_ref, out_ref,
          ctx_scr, rbuf, kf_scr, vf_scr, loc_sems,
          ss_rs, rs_rs, ss_ag, rs_ag):
    my = lax.axis_index("i")
    v = my ^ ((my >> 1) & 1)

    def phys(u):
        return u ^ ((u >> 1) & 1)

    partners = {m: phys(v ^ m) for m in (1, 2, 4)}

    barrier = pltpu.get_barrier_semaphore()
    for m in (1, 2, 4):
        pl.semaphore_signal(
            barrier, inc=1,
            device_id=(partners[m],), device_id_type=pl.DeviceIdType.MESH,
        )
    pl.semaphore_wait(barrier, 3)

    kv_copies = []
    for b in range(B):
        for h in range(HQ_LOC):
            bh = b * HQ_LOC + h
            hg = my * HQ_LOC + h
            for src, dst, si in ((k_ref, kf_scr, 0), (v_ref, vf_scr, 1)):
                cp = pltpu.make_async_copy(
                    src.at[b, :, hg, :], dst.at[bh], loc_sems.at[2 * bh + si]
                )
                cp.start()
                kv_copies.append(cp)

    qv = jnp.dot(
        x_ref[...], wq_ref[...], preferred_element_type=jnp.float32
    ).astype(jnp.bfloat16)

    for cp in kv_copies:
        cp.wait()

    biases = []
    for qi in range(SQ // QT):
        klen = (qi + 1) * QT
        rows = lax.broadcasted_iota(jnp.int32, (QT, klen), 0)
        cols = lax.broadcasted_iota(jnp.int32, (QT, klen), 1)
        biases.append(
            jnp.where((rows < 64) & (cols >= klen - 64), -1e9, 0.0)
        )

    for b in range(B):
        for h in range(HQ_LOC):
            bh = b * HQ_LOC + h
            q_bh = lax.slice(qv, (b * SQ, h * DH), ((b + 1) * SQ, (h + 1) * DH))
            k_bh = kf_scr[bh].astype(jnp.bfloat16)
            v_bh = vf_scr[bh].astype(jnp.bfloat16)
            for qi in range(SQ // QT):
                klen = (qi + 1) * QT
                q_tile = lax.slice(q_bh, (qi * QT, 0), ((qi + 1) * QT, DH))
                s = lax.dot_general(
                    q_tile, lax.slice(k_bh, (0, 0), (klen, DH)),
                    (((1,), (1,)), ((), ())),
                    preferred_element_type=jnp.float32,
                )
                s = s + biases[qi]
                m_ = jnp.max(s, axis=-1, keepdims=True)
                e = jnp.exp(s - m_)
                p = (e / jnp.sum(e, axis=-1, keepdims=True)).astype(jnp.bfloat16)
                o = jnp.dot(
                    p, lax.slice(v_bh, (0, 0), (klen, DH)),
                    preferred_element_type=jnp.float32,
                )
                ctx_scr[bh, pl.ds(qi * QT, QT), :] = o.astype(jnp.bfloat16)

    def gcols(g):
        return pl.ds(g * COLS, COLS)

    rs_idx = [0]
    ag_idx = [0]

    def issue_rs(g, j):
        fixedmask, m, runs = RS_SCHED[g][j]
        send_base = (v & fixedmask) | ((v ^ m) & m)
        descs = []
        slot = RBUF_BASE[j]
        for off, n in runs:
            i = rs_idx[0]
            rs_idx[0] += 1
            rdma = pltpu.make_async_remote_copy(
                src_ref=out_ref.at[pl.ds((send_base + off) * R, n * R), gcols(g)],
                dst_ref=rbuf.at[pl.ds(slot * R, n * R), gcols(g)],
                send_sem=ss_rs.at[i],
                recv_sem=rs_rs.at[i],
                device_id=(partners[m],),
                device_id_type=pl.DeviceIdType.MESH,
            )
            rdma.start()
            descs.append(rdma)
            slot += n
        return descs

    def add_rs(g, j):
        fixedmask, m, runs = RS_SCHED[g][j]
        keep_base = (v & fixedmask) | (v & m)
        slot = RBUF_BASE[j]
        for off, n in runs:
            rows = pl.ds((keep_base + off) * R, n * R)
            out_ref[rows, gcols(g)] = (
                out_ref[rows, gcols(g)]
                + rbuf[pl.ds(slot * R, n * R), gcols(g)]
            )
            slot += n

    def issue_ag(g, j):
        validmask, m, runs = AG_SCHED[g][j]
        base = v & validmask
        descs = []
        for off, n in runs:
            i = ag_idx[0]
            ag_idx[0] += 1
            rows = pl.ds((base + off) * R, n * R)
            rdma = pltpu.make_async_remote_copy(
                src_ref=out_ref.at[rows, gcols(g)],
                dst_ref=out_ref.at[rows, gcols(g)],
                send_sem=ss_ag.at[i],
                recv_sem=rs_ag.at[i],
                device_id=(partners[m],),
                device_id_type=pl.DeviceIdType.MESH,
            )
            rdma.start()
            descs.append(rdma)
        return descs

    half_a = ((v ^ 4) & 4) * R
    half_b = (v & 4) * R

    def project(base):
        bsel = base // (4 * R)
        ctx_cat = jnp.concatenate(
            [ctx_scr[bsel * HQ_LOC + h] for h in range(HQ_LOC)], axis=1
        )
        out_ref[pl.ds(base, 4 * R), :] = jnp.dot(
            ctx_cat, wo_ref[...], preferred_element_type=jnp.float32
        ).astype(jnp.bfloat16)

    project(half_a)
    pend = {0: issue_rs(0, 0)}
    project(half_b)
    pend[1] = issue_rs(1, 0)
    pend[2] = issue_rs(2, 0)

    ag_pend = {}
    for j in range(3):
        for g in range(NG):
            for d in pend[g]:
                d.wait()
            add_rs(g, j)
            if j < 2:
                pend[g] = issue_rs(g, j + 1)
            else:
                ag_pend[g] = issue_ag(g, 0)

    for j in range(3):
        for g in range(NG):
            for d in ag_pend[g]:
                d.wait()
            if j < 2:
                ag_pend[g] = issue_ag(g, j + 1)


def kernel(x, Wq, K_ext, V_ext, Wo):
    xb = x.astype(jnp.bfloat16).reshape(ROWS, EMB)
    wqb = (Wq * 0.125).astype(jnp.bfloat16)
    wob = Wo.astype(jnp.bfloat16)

    out = pl.pallas_call(
        _body,
        out_shape=jax.ShapeDtypeStruct((ROWS, EMB), jnp.bfloat16),
        in_specs=[
            pl.BlockSpec(memory_space=pltpu.VMEM),
            pl.BlockSpec(memory_space=pltpu.VMEM),
            pl.BlockSpec(memory_space=pltpu.MemorySpace.HBM),
            pl.BlockSpec(memory_space=pltpu.MemorySpace.HBM),
            pl.BlockSpec(memory_space=pltpu.VMEM),
        ],
        out_specs=pl.BlockSpec(memory_space=pltpu.VMEM),
        scratch_shapes=[
            pltpu.VMEM((B * HQ_LOC, SQ, DH), jnp.bfloat16),
            pltpu.VMEM((7 * R, EMB), jnp.bfloat16),
            pltpu.VMEM((B * HQ_LOC, SKV, DH), jnp.float32),
            pltpu.VMEM((B * HQ_LOC, SKV, DH), jnp.float32),
            pltpu.SemaphoreType.DMA((4 * HQ_LOC,)),
            pltpu.SemaphoreType.DMA((N_RDMA,)),
            pltpu.SemaphoreType.DMA((N_RDMA,)),
            pltpu.SemaphoreType.DMA((N_RDMA,)),
            pltpu.SemaphoreType.DMA((N_RDMA,)),
        ],
        compiler_params=pltpu.CompilerParams(collective_id=0),
    )(xb, wqb, K_ext, V_ext, wob)
    return out.reshape(B, SQ, EMB).astype(jnp.float32)


# device time: 51574 ns/iter; 2.2023x vs baseline; 2.2023x over previous
import jax
import jax.numpy as jnp
from jax import lax
from jax.experimental import pallas as pl
from jax.experimental.pallas import tpu as pltpu

N_DEV = 8
B, SQ, SKV = 2, 512, 512
HQ_LOC, DH = 8, 64
HD = HQ_LOC * DH
EMB = 768
ROWS = B * SQ
RB = SQ // N_DEV
NG = 3
COLS = EMB // NG
QT = 128

RS_SCHED = (
    ((0, 4, ((0, 4),)), (4, 2, ((0, 2),)), (6, 1, ((0, 1),))),
    ((0, 2, ((0, 2), (4, 2))), (2, 1, ((0, 1), (4, 1))), (3, 4, ((0, 1),))),
    ((0, 1, ((0, 1), (2, 1), (4, 1), (6, 1))), (1, 4, ((0, 1), (2, 1))),
     (5, 2, ((0, 1),))),
)
AG_SCHED = (
    ((7, 1, ((0, 1),)), (6, 2, ((0, 2),)), (4, 4, ((0, 4),))),
    ((7, 4, ((0, 1),)), (3, 1, ((0, 1), (4, 1))), (2, 2, ((0, 2), (4, 2)))),
    ((7, 2, ((0, 1),)), (5, 4, ((0, 1), (2, 1))),
     (1, 1, ((0, 1), (2, 1), (4, 1), (6, 1)))),
)
RBUF_BASE = (0, 4, 6)
N_RDMA_B = 15
RBUF_CH_B = 7


class _BatchAllReduce:

    def __init__(self, b, v, partners, out_ref, rbuf, sems):
        self.v = v
        self.partners = partners
        self.out = out_ref
        self.rbuf = rbuf
        self.ss_rs, self.rs_rs, self.ss_ag, self.rs_ag = sems
        self.row0 = b * SQ
        self.rbuf0 = b * RBUF_CH_B * RB
        self.sem0 = b * N_RDMA_B
        self.rs_i = 0
        self.ag_i = 0
        self.phase = 0
        self.pend = {}

    @staticmethod
    def _gcols(g):
        return pl.ds(g * COLS, COLS)

    def _issue_rs(self, g, j):
        fixedmask, m, runs = RS_SCHED[g][j]
        base = (self.v & fixedmask) | ((self.v ^ m) & m)
        descs = []
        slot = RBUF_BASE[j]
        for off, n in runs:
            i = self.sem0 + self.rs_i
            self.rs_i += 1
            rdma = pltpu.make_async_remote_copy(
                src_ref=self.out.at[
                    pl.ds(self.row0 + (base + off) * RB, n * RB), self._gcols(g)
                ],
                dst_ref=self.rbuf.at[
                    pl.ds(self.rbuf0 + slot * RB, n * RB), self._gcols(g)
                ],
                send_sem=self.ss_rs.at[i],
                recv_sem=self.rs_rs.at[i],
                device_id=(self.partners[m],),
                device_id_type=pl.DeviceIdType.MESH,
            )
            rdma.start()
            descs.append(rdma)
            slot += n
        return descs

    def _add_rs(self, g, j):
        fixedmask, m, runs = RS_SCHED[g][j]
        keep = (self.v & fixedmask) | (self.v & m)
        slot = RBUF_BASE[j]
        for off, n in runs:
            rows = pl.ds(self.row0 + (keep + off) * RB, n * RB)
            self.out[rows, self._gcols(g)] = (
                self.out[rows, self._gcols(g)]
                + self.rbuf[pl.ds(self.rbuf0 + slot * RB, n * RB), self._gcols(g)]
            )
            slot += n

    def _issue_ag(self, g, j):
        validmask, m, runs = AG_SCHED[g][j]
        base = self.v & validmask
        descs = []
        for off, n in runs:
            i = self.sem0 + self.ag_i
            self.ag_i += 1
            rows = pl.ds(self.row0 + (base + off) * RB, n * RB)
            rdma = pltpu.make_async_remote_copy(
                src_ref=self.out.at[rows, self._gcols(g)],
                dst_ref=self.out.at[rows, self._gcols(g)],
                send_sem=self.ss_ag.at[i],
                recv_sem=self.rs_ag.at[i],
                device_id=(self.partners[m],),
                device_id_type=pl.DeviceIdType.MESH,
            )
            rdma.start()
            descs.append(rdma)
        return descs

    def start(self):
        self.pend = {g: self._issue_rs(g, 0) for g in range(NG)}

    def service(self):
        j = self.phase
        for g in range(NG):
            for d in self.pend[g]:
                d.wait()
            if j < 3:
                self._add_rs(g, j)
            if j < 2:
                self.pend[g] = self._issue_rs(g, j + 1)
            elif j < 5:
                self.pend[g] = self._issue_ag(g, j - 2)
            else:
                self.pend[g] = []
        self.phase += 1


def _body(x_ref, wq_ref, k_ref, v_ref, wo_ref, out_ref,
          ctx_scr, rbuf, ss_rs, rs_rs, ss_ag, rs_ag):
    my = lax.axis_index("i")
    v = my ^ ((my >> 1) & 1)

    def phys(u):
        return u ^ ((u >> 1) & 1)

    partners = {m: phys(v ^ m) for m in (1, 2, 4)}

    barrier = pltpu.get_barrier_semaphore()
    for m in (1, 2, 4):
        pl.semaphore_signal(
            barrier, inc=1,
            device_id=(partners[m],), device_id_type=pl.DeviceIdType.MESH,
        )
    pl.semaphore_wait(barrier, 3)

    qv = jnp.dot(
        x_ref[...], wq_ref[...], preferred_element_type=jnp.float32
    ).astype(jnp.bfloat16)

    biases = []
    for qi in range(SQ // QT):
        klen = (qi + 1) * QT
        rows = lax.broadcasted_iota(jnp.int32, (QT, klen), 0)
        cols = lax.broadcasted_iota(jnp.int32, (QT, klen), 1)
        biases.append(jnp.where((rows < 64) & (cols >= klen - 64), -1e9, 0.0))

    def attn_head(b, h):
        bh = b * HQ_LOC + h
        q_bh = lax.slice(qv, (b * SQ, h * DH), ((b + 1) * SQ, (h + 1) * DH))
        for qi in range(SQ // QT):
            klen = (qi + 1) * QT
            q_tile = lax.slice(q_bh, (qi * QT, 0), ((qi + 1) * QT, DH))
            s = lax.dot_general(
                q_tile, k_ref[bh, pl.ds(0, klen), :],
                (((1,), (1,)), ((), ())),
                preferred_element_type=jnp.float32,
            )
            s = s + biases[qi]
            m_ = jnp.max(s, axis=-1, keepdims=True)
            e = jnp.exp(s - m_)
            p = (e / jnp.sum(e, axis=-1, keepdims=True)).astype(jnp.bfloat16)
            o = jnp.dot(
                p, v_ref[bh, pl.ds(0, klen), :],
                preferred_element_type=jnp.float32,
            )
            ctx_scr[bh, pl.ds(qi * QT, QT), :] = o.astype(jnp.bfloat16)

    def wo_proj(b):
        ctx_cat = jnp.concatenate(
            [ctx_scr[b * HQ_LOC + h] for h in range(HQ_LOC)], axis=1
        )
        out_ref[pl.ds(b * SQ, SQ), :] = jnp.dot(
            ctx_cat, wo_ref[...], preferred_element_type=jnp.float32
        ).astype(jnp.bfloat16)

    sems = (ss_rs, rs_rs, ss_ag, rs_ag)
    ar0 = _BatchAllReduce(0, v, partners, out_ref, rbuf, sems)
    ar1 = _BatchAllReduce(1, v, partners, out_ref, rbuf, sems)

    for h in range(HQ_LOC):
        attn_head(0, h)
    wo_proj(0)
    ar0.start()

    for h in range(HQ_LOC):
        attn_head(1, h)
        if h % 2 == 1:
            ar0.service()
    wo_proj(1)
    ar1.start()
    ar0.service()
    ar1.service()
    ar0.service()
    for _ in range(5):
        ar1.service()


def kernel(x, Wq, K_ext, V_ext, Wo):
    my = lax.axis_index("i")

    xb = x.astype(jnp.bfloat16).reshape(ROWS, EMB)
    wqb = (Wq * 0.125).astype(jnp.bfloat16)
    wob = Wo.astype(jnp.bfloat16)
    k = lax.dynamic_slice_in_dim(K_ext, my * HQ_LOC, HQ_LOC, axis=2)
    v = lax.dynamic_slice_in_dim(V_ext, my * HQ_LOC, HQ_LOC, axis=2)
    k8 = k.astype(jnp.bfloat16).transpose(0, 2, 1, 3).reshape(B * HQ_LOC, SKV, DH)
    v8 = v.astype(jnp.bfloat16).transpose(0, 2, 1, 3).reshape(B * HQ_LOC, SKV, DH)

    out = pl.pallas_call(
        _body,
        out_shape=jax.ShapeDtypeStruct((ROWS, EMB), jnp.bfloat16),
        in_specs=[pl.BlockSpec(memory_space=pltpu.VMEM)] * 5,
        out_specs=pl.BlockSpec(memory_space=pltpu.VMEM),
        scratch_shapes=[
            pltpu.VMEM((B * HQ_LOC, SQ, DH), jnp.bfloat16),
            pltpu.VMEM((B * RBUF_CH_B * RB, EMB), jnp.bfloat16),
            pltpu.SemaphoreType.DMA((B * N_RDMA_B,)),
            pltpu.SemaphoreType.DMA((B * N_RDMA_B,)),
            pltpu.SemaphoreType.DMA((B * N_RDMA_B,)),
            pltpu.SemaphoreType.DMA((B * N_RDMA_B,)),
        ],
        compiler_params=pltpu.CompilerParams(collective_id=0),
    )(xb, wqb, k8, v8, wob)
    return out.reshape(B, SQ, EMB).astype(jnp.float32)


# device time: 48451 ns/iter; 2.3443x vs baseline; 1.0645x over previous
import jax
import jax.numpy as jnp
from jax import lax
from jax.experimental import pallas as pl
from jax.experimental.pallas import tpu as pltpu

N_DEV = 8
B, SQ, SKV = 2, 512, 512
HQ_LOC, DH = 8, 64
HD = HQ_LOC * DH
EMB = 768
ROWS = B * SQ
RB = SQ // N_DEV
NG = 3
COLS = EMB // NG
QT = 256

RS_SCHED = (
    ((0, 4, ((0, 4),)), (4, 2, ((0, 2),)), (6, 1, ((0, 1),))),
    ((0, 2, ((0, 2), (4, 2))), (2, 1, ((0, 1), (4, 1))), (3, 4, ((0, 1),))),
    ((0, 1, ((0, 1), (2, 1), (4, 1), (6, 1))), (1, 4, ((0, 1), (2, 1))),
     (5, 2, ((0, 1),))),
)
AG_SCHED = (
    ((7, 1, ((0, 1),)), (6, 2, ((0, 2),)), (4, 4, ((0, 4),))),
    ((7, 4, ((0, 1),)), (3, 1, ((0, 1), (4, 1))), (2, 2, ((0, 2), (4, 2)))),
    ((7, 2, ((0, 1),)), (5, 4, ((0, 1), (2, 1))),
     (1, 1, ((0, 1), (2, 1), (4, 1), (6, 1)))),
)
RBUF_BASE = (0, 4, 6)
N_RDMA_B = 15
RBUF_CH_B = 7


class _BatchAllReduce:

    def __init__(self, b, v, partners, out_ref, rbuf, sems):
        self.v = v
        self.partners = partners
        self.out = out_ref
        self.rbuf = rbuf
        self.ss_rs, self.rs_rs, self.ss_ag, self.rs_ag = sems
        self.row0 = b * SQ
        self.rbuf0 = b * RBUF_CH_B * RB
        self.sem0 = b * N_RDMA_B
        self.rs_i = 0
        self.ag_i = 0
        self.phase = 0
        self.pend = {}

    @staticmethod
    def _gcols(g):
        return pl.ds(g * COLS, COLS)

    def _issue_rs(self, g, j):
        fixedmask, m, runs = RS_SCHED[g][j]
        base = (self.v & fixedmask) | ((self.v ^ m) & m)
        descs = []
        slot = RBUF_BASE[j]
        for off, n in runs:
            i = self.sem0 + self.rs_i
            self.rs_i += 1
            rdma = pltpu.make_async_remote_copy(
                src_ref=self.out.at[
                    pl.ds(self.row0 + (base + off) * RB, n * RB), self._gcols(g)
                ],
                dst_ref=self.rbuf.at[
                    pl.ds(self.rbuf0 + slot * RB, n * RB), self._gcols(g)
                ],
                send_sem=self.ss_rs.at[i],
                recv_sem=self.rs_rs.at[i],
                device_id=(self.partners[m],),
                device_id_type=pl.DeviceIdType.MESH,
            )
            rdma.start()
            descs.append(rdma)
            slot += n
        return descs

    def _add_rs(self, g, j):
        fixedmask, m, runs = RS_SCHED[g][j]
        keep = (self.v & fixedmask) | (self.v & m)
        slot = RBUF_BASE[j]
        for off, n in runs:
            rows = pl.ds(self.row0 + (keep + off) * RB, n * RB)
            self.out[rows, self._gcols(g)] = (
                self.out[rows, self._gcols(g)]
                + self.rbuf[pl.ds(self.rbuf0 + slot * RB, n * RB), self._gcols(g)]
            )
            slot += n

    def _issue_ag(self, g, j):
        validmask, m, runs = AG_SCHED[g][j]
        base = self.v & validmask
        descs = []
        for off, n in runs:
            i = self.sem0 + self.ag_i
            self.ag_i += 1
            rows = pl.ds(self.row0 + (base + off) * RB, n * RB)
            rdma = pltpu.make_async_remote_copy(
                src_ref=self.out.at[rows, self._gcols(g)],
                dst_ref=self.out.at[rows, self._gcols(g)],
                send_sem=self.ss_ag.at[i],
                recv_sem=self.rs_ag.at[i],
                device_id=(self.partners[m],),
                device_id_type=pl.DeviceIdType.MESH,
            )
            rdma.start()
            descs.append(rdma)
        return descs

    def start(self):
        self.pend = {g: self._issue_rs(g, 0) for g in range(NG)}

    def service(self):
        j = self.phase
        for g in range(NG):
            for d in self.pend[g]:
                d.wait()
            if j < 3:
                self._add_rs(g, j)
            if j < 2:
                self.pend[g] = self._issue_rs(g, j + 1)
            elif j < 5:
                self.pend[g] = self._issue_ag(g, j - 2)
            else:
                self.pend[g] = []
        self.phase += 1


def _body(x_ref, wq_ref, k_ref, v_ref, wo_ref, out_ref,
          ctx_scr, rbuf, ss_rs, rs_rs, ss_ag, rs_ag):
    my = lax.axis_index("i")
    v = my ^ ((my >> 1) & 1)

    def phys(u):
        return u ^ ((u >> 1) & 1)

    partners = {m: phys(v ^ m) for m in (1, 2, 4)}

    barrier = pltpu.get_barrier_semaphore()
    for m in (1, 2, 4):
        pl.semaphore_signal(
            barrier, inc=1,
            device_id=(partners[m],), device_id_type=pl.DeviceIdType.MESH,
        )
    pl.semaphore_wait(barrier, 3)

    qv = jnp.dot(
        x_ref[...], wq_ref[...], preferred_element_type=jnp.float32
    ).astype(jnp.bfloat16)

    biases = []
    for qi in range(SQ // QT):
        klen = (qi + 1) * QT
        rows = lax.broadcasted_iota(jnp.int32, (QT, klen), 0)
        cols = lax.broadcasted_iota(jnp.int32, (QT, klen), 1)
        biases.append(
            jnp.where(cols - (rows // 64) * 64 >= klen - QT + 64, -1e9, 0.0)
        )

    def attn_head(b, h):
        bh = b * HQ_LOC + h
        q_bh = lax.slice(qv, (b * SQ, h * DH), ((b + 1) * SQ, (h + 1) * DH))
        for qi in range(SQ // QT):
            klen = (qi + 1) * QT
            q_tile = lax.slice(q_bh, (qi * QT, 0), ((qi + 1) * QT, DH))
            s = lax.dot_general(
                q_tile, k_ref[bh, pl.ds(0, klen), :],
                (((1,), (1,)), ((), ())),
                preferred_element_type=jnp.float32,
            )
            s = s + biases[qi]
            m_ = jnp.max(s, axis=-1, keepdims=True)
            e = jnp.exp(s - m_)
            p = (e / jnp.sum(e, axis=-1, keepdims=True)).astype(jnp.bfloat16)
            o = jnp.dot(
                p, v_ref[bh, pl.ds(0, klen), :],
                preferred_element_type=jnp.float32,
            )
            ctx_scr[bh, pl.ds(qi * QT, QT), :] = o.astype(jnp.bfloat16)

    def wo_proj(b):
        ctx_cat = jnp.concatenate(
            [ctx_scr[b * HQ_LOC + h] for h in range(HQ_LOC)], axis=1
        )
        out_ref[pl.ds(b * SQ, SQ), :] = jnp.dot(
            ctx_cat, wo_ref[...], preferred_element_type=jnp.float32
        ).astype(jnp.bfloat16)

    sems = (ss_rs, rs_rs, ss_ag, rs_ag)
    ar0 = _BatchAllReduce(0, v, partners, out_ref, rbuf, sems)
    ar1 = _BatchAllReduce(1, v, partners, out_ref, rbuf, sems)

    for h in range(HQ_LOC):
        attn_head(0, h)
    wo_proj(0)
    ar0.start()

    for h in range(HQ_LOC):
        attn_head(1, h)
        if h % 2 == 1:
            ar0.service()
    wo_proj(1)
    ar1.start()
    ar0.service()
    ar1.service()
    ar0.service()
    for _ in range(5):
        ar1.service()


def kernel(x, Wq, K_ext, V_ext, Wo):
    my = lax.axis_index("i")

    xb = x.astype(jnp.bfloat16).reshape(ROWS, EMB)
    wqb = (Wq * 0.125).astype(jnp.bfloat16)
    wob = Wo.astype(jnp.bfloat16)
    k = lax.dynamic_slice_in_dim(K_ext, my * HQ_LOC, HQ_LOC, axis=2)
    v = lax.dynamic_slice_in_dim(V_ext, my * HQ_LOC, HQ_LOC, axis=2)
    k8 = k.astype(jnp.bfloat16).transpose(0, 2, 1, 3).reshape(B * HQ_LOC, SKV, DH)
    v8 = v.astype(jnp.bfloat16).transpose(0, 2, 1, 3).reshape(B * HQ_LOC, SKV, DH)

    out = pl.pallas_call(
        _body,
        out_shape=jax.ShapeDtypeStruct((ROWS, EMB), jnp.bfloat16),
        in_specs=[pl.BlockSpec(memory_space=pltpu.VMEM)] * 5,
        out_specs=pl.BlockSpec(memory_space=pltpu.VMEM),
        scratch_shapes=[
            pltpu.VMEM((B * HQ_LOC, SQ, DH), jnp.bfloat16),
            pltpu.VMEM((B * RBUF_CH_B * RB, EMB), jnp.bfloat16),
            pltpu.SemaphoreType.DMA((B * N_RDMA_B,)),
            pltpu.SemaphoreType.DMA((B * N_RDMA_B,)),
            pltpu.SemaphoreType.DMA((B * N_RDMA_B,)),
            pltpu.SemaphoreType.DMA((B * N_RDMA_B,)),
        ],
        compiler_params=pltpu.CompilerParams(collective_id=0),
    )(xb, wqb, k8, v8, wob)
    return out.reshape(B, SQ, EMB).astype(jnp.float32)


# device time: 47666 ns/iter; 2.3829x vs baseline; 1.0165x over previous
import jax
import jax.numpy as jnp
from jax import lax
from jax.experimental import pallas as pl
from jax.experimental.pallas import tpu as pltpu

N_DEV = 8
B, SQ, SKV = 2, 512, 512
HQ_LOC, DH = 8, 64
HD = HQ_LOC * DH
EMB = 768
ROWS = B * SQ
RB = SQ // N_DEV
NG = 3
COLS = EMB // NG
QT = 256

RS_SCHED = (
    ((0, 4, ((0, 4),)), (4, 2, ((0, 2),)), (6, 1, ((0, 1),))),
    ((0, 2, ((0, 2), (4, 2))), (2, 1, ((0, 1), (4, 1))), (3, 4, ((0, 1),))),
    ((0, 1, ((0, 1), (2, 1), (4, 1), (6, 1))), (1, 4, ((0, 1), (2, 1))),
     (5, 2, ((0, 1),))),
)
AG_SCHED = (
    ((7, 1, ((0, 1),)), (6, 2, ((0, 2),)), (4, 4, ((0, 4),))),
    ((7, 4, ((0, 1),)), (3, 1, ((0, 1), (4, 1))), (2, 2, ((0, 2), (4, 2)))),
    ((7, 2, ((0, 1),)), (5, 4, ((0, 1), (2, 1))),
     (1, 1, ((0, 1), (2, 1), (4, 1), (6, 1)))),
)
RBUF_BASE = (0, 4, 6)
N_RDMA_B = 15
RBUF_CH_B = 7


class _BatchAllReduce:

    def __init__(self, b, v, partners, out_ref, rbuf, sems):
        self.v = v
        self.partners = partners
        self.out = out_ref
        self.rbuf = rbuf
        self.ss_rs, self.rs_rs, self.ss_ag, self.rs_ag = sems
        self.row0 = b * SQ
        self.rbuf0 = b * RBUF_CH_B * RB
        self.sem0 = b * N_RDMA_B
        self.rs_i = 0
        self.ag_i = 0
        self.phase = 0
        self.pend = {}

    @staticmethod
    def _gcols(g):
        return pl.ds(g * COLS, COLS)

    def _issue_rs(self, g, j):
        fixedmask, m, runs = RS_SCHED[g][j]
        base = (self.v & fixedmask) | ((self.v ^ m) & m)
        descs = []
        slot = RBUF_BASE[j]
        for off, n in runs:
            i = self.sem0 + self.rs_i
            self.rs_i += 1
            rdma = pltpu.make_async_remote_copy(
                src_ref=self.out.at[
                    pl.ds(self.row0 + (base + off) * RB, n * RB), self._gcols(g)
                ],
                dst_ref=self.rbuf.at[
                    pl.ds(self.rbuf0 + slot * RB, n * RB), self._gcols(g)
                ],
                send_sem=self.ss_rs.at[i],
                recv_sem=self.rs_rs.at[i],
                device_id=(self.partners[m],),
                device_id_type=pl.DeviceIdType.MESH,
            )
            rdma.start()
            descs.append(rdma)
            slot += n
        return descs

    def _add_rs(self, g, j):
        fixedmask, m, runs = RS_SCHED[g][j]
        keep = (self.v & fixedmask) | (self.v & m)
        slot = RBUF_BASE[j]
        for off, n in runs:
            rows = pl.ds(self.row0 + (keep + off) * RB, n * RB)
            self.out[rows, self._gcols(g)] = (
                self.out[rows, self._gcols(g)]
                + self.rbuf[pl.ds(self.rbuf0 + slot * RB, n * RB), self._gcols(g)]
            )
            slot += n

    def _issue_ag(self, g, j):
        validmask, m, runs = AG_SCHED[g][j]
        base = self.v & validmask
        descs = []
        for off, n in runs:
            i = self.sem0 + self.ag_i
            self.ag_i += 1
            rows = pl.ds(self.row0 + (base + off) * RB, n * RB)
            rdma = pltpu.make_async_remote_copy(
                src_ref=self.out.at[rows, self._gcols(g)],
                dst_ref=self.out.at[rows, self._gcols(g)],
                send_sem=self.ss_ag.at[i],
                recv_sem=self.rs_ag.at[i],
                device_id=(self.partners[m],),
                device_id_type=pl.DeviceIdType.MESH,
            )
            rdma.start()
            descs.append(rdma)
        return descs

    def start(self):
        self.pend = {g: self._issue_rs(g, 0) for g in range(NG)}

    def service(self):
        j = self.phase
        for g in range(NG):
            for d in self.pend[g]:
                d.wait()
            if j < 3:
                self._add_rs(g, j)
            if j < 2:
                self.pend[g] = self._issue_rs(g, j + 1)
            elif j < 5:
                self.pend[g] = self._issue_ag(g, j - 2)
            else:
                self.pend[g] = []
        self.phase += 1


def _body(x_ref, wq_ref, k_ref, v_ref, wo_ref, out_ref,
          ctx_scr, rbuf, ss_rs, rs_rs, ss_ag, rs_ag):
    my = lax.axis_index("i")
    v = my ^ ((my >> 1) & 1)

    def phys(u):
        return u ^ ((u >> 1) & 1)

    partners = {m: phys(v ^ m) for m in (1, 2, 4)}

    barrier = pltpu.get_barrier_semaphore()
    for m in (1, 2, 4):
        pl.semaphore_signal(
            barrier, inc=1,
            device_id=(partners[m],), device_id_type=pl.DeviceIdType.MESH,
        )
    pl.semaphore_wait(barrier, 3)

    qv = jnp.dot(
        x_ref[...], wq_ref[...], preferred_element_type=jnp.float32
    ).astype(jnp.bfloat16)

    biases = []
    for qi in range(SQ // QT):
        klen = (qi + 1) * QT
        rows = lax.broadcasted_iota(jnp.int32, (QT, klen), 0)
        cols = lax.broadcasted_iota(jnp.int32, (QT, klen), 1)
        biases.append(
            jnp.where(cols - (rows // 64) * 64 >= klen - QT + 64, -1e9, 0.0)
        )

    def attn_head(b, h):
        bh = b * HQ_LOC + h
        q_bh = lax.slice(qv, (b * SQ, h * DH), ((b + 1) * SQ, (h + 1) * DH))
        for qi in range(SQ // QT):
            klen = (qi + 1) * QT
            q_tile = lax.slice(q_bh, (qi * QT, 0), ((qi + 1) * QT, DH))
            s = lax.dot_general(
                q_tile, k_ref[bh, pl.ds(0, klen), :],
                (((1,), (1,)), ((), ())),
                preferred_element_type=jnp.float32,
            )
            s = s + biases[qi]
            m_ = jnp.max(s, axis=-1, keepdims=True)
            e = jnp.exp(s - m_)
            p = (e / jnp.sum(e, axis=-1, keepdims=True)).astype(jnp.bfloat16)
            o = jnp.dot(
                p, v_ref[bh, pl.ds(0, klen), :],
                preferred_element_type=jnp.float32,
            )
            ctx_scr[bh, pl.ds(qi * QT, QT), :] = o.astype(jnp.bfloat16)

    def wo_proj(b):
        ctx_cat = jnp.concatenate(
            [ctx_scr[b * HQ_LOC + h] for h in range(HQ_LOC)], axis=1
        )
        out_ref[pl.ds(b * SQ, SQ), :] = jnp.dot(
            ctx_cat, wo_ref[...], preferred_element_type=jnp.float32
        ).astype(jnp.bfloat16)

    sems = (ss_rs, rs_rs, ss_ag, rs_ag)
    ar0 = _BatchAllReduce(0, v, partners, out_ref, rbuf, sems)
    ar1 = _BatchAllReduce(1, v, partners, out_ref, rbuf, sems)

    for h in range(HQ_LOC):
        attn_head(0, h)
    wo_proj(0)
    ar0.start()

    for h in range(HQ_LOC):
        attn_head(1, h)
        if h % 2 == 1:
            ar0.service()
    wo_proj(1)
    ar1.start()
    ar0.service()
    ar1.service()
    ar0.service()
    for _ in range(5):
        ar1.service()


def kernel(x, Wq, K_ext, V_ext, Wo):
    my = lax.axis_index("i")

    xb = x.astype(jnp.bfloat16).reshape(ROWS, EMB)
    wqb = (Wq * 0.125).astype(jnp.bfloat16)
    wob = Wo.astype(jnp.bfloat16)
    k = lax.dynamic_slice_in_dim(K_ext, my * HQ_LOC, HQ_LOC, axis=2)
    v = lax.dynamic_slice_in_dim(V_ext, my * HQ_LOC, HQ_LOC, axis=2)
    k8 = k.astype(jnp.bfloat16).transpose(0, 2, 1, 3).reshape(B * HQ_LOC, SKV, DH)
    v8 = v.astype(jnp.bfloat16).transpose(0, 2, 1, 3).reshape(B * HQ_LOC, SKV, DH)

    out = pl.pallas_call(
        _body,
        out_shape=jax.ShapeDtypeStruct((ROWS, EMB), jnp.bfloat16),
        in_specs=[pl.BlockSpec(memory_space=pltpu.VMEM)] * 5,
        out_specs=pl.BlockSpec(memory_space=pltpu.VMEM),
        scratch_shapes=[
            pltpu.VMEM((B * HQ_LOC, SQ, DH), jnp.bfloat16),
            pltpu.VMEM((B * RBUF_CH_B * RB, EMB), jnp.bfloat16),
            pltpu.SemaphoreType.DMA((B * N_RDMA_B,)),
            pltpu.SemaphoreType.DMA((B * N_RDMA_B,)),
            pltpu.SemaphoreType.DMA((B * N_RDMA_B,)),
            pltpu.SemaphoreType.DMA((B * N_RDMA_B,)),
        ],
        compiler_params=pltpu.CompilerParams(collective_id=0),
    )(xb, wqb, k8, v8, wob)
    return out.reshape(B, SQ, EMB)
